# baseline (speedup 1.0000x reference)
import functools
import jax
import jax.numpy as jnp
from jax import lax
from jax.experimental import pallas as pl
from jax.experimental.pallas import tpu as pltpu
from jax.experimental.pallas import tpu_sc as plsc

D = 64
NC = 2
NS = 16
NW = NC * NS
CHUNK = 128
NBUF = 8


@functools.cache
def _make_kernel(n_lookups):
    n_chunks = n_lookups // CHUNK
    assert n_lookups % CHUNK == 0 and n_chunks % NW == 0
    cpw = n_chunks // NW
    assert cpw % NBUF == 0
    nsteps = cpw // NBUF
    mesh = plsc.VectorSubcoreMesh(
        core_axis_name="c", subcore_axis_name="s",
        num_cores=NC, num_subcores=NS)

    @functools.partial(
        pl.kernel,
        out_type=jax.ShapeDtypeStruct((n_lookups, D), jnp.float32),
        mesh=mesh,
        scratch_types=(
            [pltpu.VMEM((cpw, CHUNK), jnp.int32)]
            + [pltpu.VMEM((CHUNK, D), jnp.float32) for _ in range(NBUF)]
            + [pltpu.SemaphoreType.DMA for _ in range(2 * NBUF)]
        ),
        compiler_params=pltpu.CompilerParams(use_tc_tiling_on_sc=False),
    )
    def gather_kernel(idx_hbm, table_hbm, out_hbm, idx_all, *bufs_sems):
        rows = bufs_sems[:NBUF]
        gsem = bufs_sems[NBUF:2 * NBUF]
        osem = bufs_sems[2 * NBUF:]
        wid = lax.axis_index("s") * NC + lax.axis_index("c")
        chunk0 = wid * cpw

        pltpu.sync_copy(idx_hbm.at[pl.ds(chunk0, cpw)], idx_all)

        def fire(i, b):
            pltpu.async_copy(table_hbm.at[idx_all.at[i]], rows[b], gsem[b])

        def wait_gather(b):
            pltpu.make_async_copy(table_hbm.at[idx_all.at[0]],
                                  rows[b], gsem[b]).wait()

        def wb_start(i, b):
            pltpu.async_copy(rows[b],
                             out_hbm.at[pl.ds((chunk0 + i) * CHUNK, CHUNK)],
                             osem[b])

        def wait_wb(b):
            pltpu.make_async_copy(rows[b], out_hbm.at[pl.ds(0, CHUNK)],
                                  osem[b]).wait()

        for b in range(NBUF):
            fire(b, b)

        @pl.loop(0, nsteps)
        def body(s):
            i0 = s * NBUF
            for b in range(NBUF):
                wait_gather(b)
                wb_start(i0 + b, b)

            @pl.when(s < nsteps - 1)
            def _():
                for b in range(NBUF):
                    wait_wb(b)
                    fire(i0 + NBUF + b, b)

        for b in range(NBUF):
            wait_wb(b)

    return gather_kernel


@functools.cache
def _make_tc_transpose(batch, n_fields):
    bh = batch // CHUNK
    fp = n_fields // 2

    def body(x_ref, y_ref):
        eye = jnp.eye(CHUNK, dtype=jnp.float32)
        x4 = x_ref[0].reshape(CHUNK, fp, 2, D)
        for f in range(n_fields):
            blk = x4[:, f // 2, f % 2, :]
            t = lax.dot_general(blk, eye, (((0,), (0,)), ((), ())),
                                preferred_element_type=jnp.float32)
            y_ref[f, :, 0, :, :] = t.reshape(D // 8, 8, CHUNK)

    return pl.pallas_call(
        body,
        grid=(bh,),
        in_specs=[pl.BlockSpec((1, n_fields * D, CHUNK),
                               lambda i: (i, 0, 0))],
        out_specs=pl.BlockSpec((n_fields, D // 8, 1, 8, CHUNK),
                               lambda i: (0, 0, i, 0, 0)),
        out_shape=jax.ShapeDtypeStruct(
            (n_fields, D // 8, bh, 8, CHUNK), jnp.float32),
    )


@functools.cache
def _make_tc_repack(vocab, bv):
    def body(x_ref, y_ref):
        eye = jnp.eye(D, dtype=jnp.float32)
        xt = lax.dot_general(x_ref[...], eye, (((0,), (0,)), ((), ())),
                             preferred_element_type=jnp.float32)
        x3 = xt.reshape(bv // 2, 2, D)
        y_ref[...] = jnp.concatenate([x3[:, 0, :], x3[:, 1, :]], axis=1)

    return pl.pallas_call(
        body,
        grid=(pl.cdiv(vocab, bv),),
        in_specs=[pl.BlockSpec((D, bv), lambda i: (0, i))],
        out_specs=pl.BlockSpec((bv // 2, 2 * D), lambda i: (i, 0)),
        out_shape=jax.ShapeDtypeStruct((vocab // 2, 2 * D), jnp.float32),
    )


def kernel(nodes, table):
    batch, n_fields = nodes.shape
    vocab = table.shape[0]
    n_lookups = batch * n_fields
    idx = nodes.reshape(n_lookups // CHUNK, CHUNK)
    packed = _make_tc_repack(vocab, 8192)(table.T)
    tlin = packed.reshape(vocab, D)
    out = _make_kernel(n_lookups)(idx, tlin)
    x3 = out.reshape(batch // CHUNK, n_fields * D, CHUNK)
    out5 = _make_tc_transpose(batch, n_fields)(x3)
    return out5.transpose(2, 4, 0, 1, 3).reshape(batch, n_fields, D)


# speedup vs baseline: 1.0764x; 1.0764x over previous
import functools
import jax
import jax.numpy as jnp
from jax import lax
from jax.experimental import pallas as pl
from jax.experimental.pallas import tpu as pltpu
from jax.experimental.pallas import tpu_sc as plsc

D = 64
NC = 2
NS = 16
NW = NC * NS
CHUNK = 128
NBUF = 8


@functools.cache
def _make_kernel(n_lookups):
    n_chunks = n_lookups // CHUNK
    assert n_lookups % CHUNK == 0 and n_chunks % NW == 0
    cpw = n_chunks // NW
    assert cpw % NBUF == 0
    nsteps = cpw // NBUF
    mesh = plsc.VectorSubcoreMesh(
        core_axis_name="c", subcore_axis_name="s",
        num_cores=NC, num_subcores=NS)

    @functools.partial(
        pl.kernel,
        out_type=jax.ShapeDtypeStruct((n_lookups, D), jnp.float32),
        mesh=mesh,
        scratch_types=(
            [pltpu.VMEM((cpw, CHUNK), jnp.int32)]
            + [pltpu.VMEM((CHUNK, D), jnp.float32) for _ in range(NBUF)]
            + [pltpu.SemaphoreType.DMA for _ in range(2 * NBUF)]
        ),
        compiler_params=pltpu.CompilerParams(use_tc_tiling_on_sc=False),
    )
    def gather_kernel(idx_hbm, table_hbm, out_hbm, idx_all, *bufs_sems):
        rows = bufs_sems[:NBUF]
        gsem = bufs_sems[NBUF:2 * NBUF]
        osem = bufs_sems[2 * NBUF:]
        wid = lax.axis_index("s") * NC + lax.axis_index("c")
        chunk0 = wid * cpw

        pltpu.sync_copy(idx_hbm.at[pl.ds(chunk0, cpw)], idx_all)

        def fire(i, b):
            pltpu.async_copy(table_hbm.at[idx_all.at[i]], rows[b], gsem[b])

        def wait_gather(b):
            pltpu.make_async_copy(table_hbm.at[idx_all.at[0]],
                                  rows[b], gsem[b]).wait()

        def wb_start(i, b):
            pltpu.async_copy(rows[b],
                             out_hbm.at[pl.ds((chunk0 + i) * CHUNK, CHUNK)],
                             osem[b])

        def wait_wb(b):
            pltpu.make_async_copy(rows[b], out_hbm.at[pl.ds(0, CHUNK)],
                                  osem[b]).wait()

        for b in range(NBUF):
            fire(b, b)

        @pl.loop(0, nsteps)
        def body(s):
            i0 = s * NBUF
            for b in range(NBUF):
                wait_gather(b)
                wb_start(i0 + b, b)

            @pl.when(s < nsteps - 1)
            def _():
                for b in range(NBUF):
                    wait_wb(b)
                    fire(i0 + NBUF + b, b)

        for b in range(NBUF):
            wait_wb(b)

    return gather_kernel


@functools.cache
def _make_tc_transpose(batch, n_fields):
    bh = batch // CHUNK
    fp = n_fields // 2

    def body(x_ref, y_ref):
        x4 = x_ref[0].reshape(CHUNK, fp, 2, D)
        for f in range(n_fields):
            blk = x4[:, f // 2, f % 2, :]
            y_ref[f, :, 0, :, :] = blk.T.reshape(D // 8, 8, CHUNK)

    return pl.pallas_call(
        body,
        grid=(bh,),
        in_specs=[pl.BlockSpec((1, n_fields * D, CHUNK),
                               lambda i: (i, 0, 0))],
        out_specs=pl.BlockSpec((n_fields, D // 8, 1, 8, CHUNK),
                               lambda i: (0, 0, i, 0, 0)),
        out_shape=jax.ShapeDtypeStruct(
            (n_fields, D // 8, bh, 8, CHUNK), jnp.float32),
    )


@functools.cache
def _make_tc_repack(vocab, bv):
    def body(x_ref, y_ref):
        xt = x_ref[...].T.reshape(bv // 2, 2, D)
        y_ref[...] = jnp.concatenate([xt[:, 0, :], xt[:, 1, :]], axis=1)

    return pl.pallas_call(
        body,
        grid=(pl.cdiv(vocab, bv),),
        in_specs=[pl.BlockSpec((D, bv), lambda i: (0, i))],
        out_specs=pl.BlockSpec((bv // 2, 2 * D), lambda i: (i, 0)),
        out_shape=jax.ShapeDtypeStruct((vocab // 2, 2 * D), jnp.float32),
    )


def kernel(nodes, table):
    batch, n_fields = nodes.shape
    vocab = table.shape[0]
    n_lookups = batch * n_fields
    idx = nodes.reshape(n_lookups // CHUNK, CHUNK)
    packed = _make_tc_repack(vocab, 16384)(table.T)
    tlin = packed.reshape(vocab, D)
    out = _make_kernel(n_lookups)(idx, tlin)
    x3 = out.reshape(batch // CHUNK, n_fields * D, CHUNK)
    out5 = _make_tc_transpose(batch, n_fields)(x3)
    return out5.transpose(2, 4, 0, 1, 3).reshape(batch, n_fields, D)
